# Initial kernel scaffold; baseline (speedup 1.0000x reference)
import jax
import jax.numpy as jnp
from jax.experimental import pallas as pl


def kernel(x, edge_index, W1, b1, W2, b2, W3, b3):
    raise NotImplementedError("write your pallas kernel here")


# speedup vs baseline: 15.2678x; 15.2678x over previous
import functools

import jax
import jax.numpy as jnp
from jax import lax
from jax.experimental import pallas as pl
from jax.experimental.pallas import tpu as pltpu
from jax.experimental.pallas import tpu_sc as plsc

N = 10000
E = 160000
IN_DIM = 256
HID = 512
OUT_DIM = 256

NC, NS = 2, 16
NW = NC * NS
EPW = E // NW
K = 100
NB = EPW // K
NP = 10240
RPT = NP // NS
BN = 1000
GRID = N // BN

_MESH = plsc.VectorSubcoreMesh(
    core_axis_name="c", subcore_axis_name="s", num_cores=NC, num_subcores=NS)


@functools.partial(
    pl.kernel,
    out_type=jax.ShapeDtypeStruct((NC, NP, 128), jnp.float32),
    mesh=_MESH,
    name="sc_degree",
    scratch_types=[
        pltpu.VMEM((NB, K), jnp.int32),
        pltpu.VMEM((K, 128), jnp.float32),
        pltpu.VMEM_SHARED((NP, 128), jnp.float32),
    ],
)
def _deg_kernel(dst_hbm, ones_hbm, zeros_hbm, out_hbm, dst_v, ones_v, acc):
    c = lax.axis_index("c")
    s = lax.axis_index("s")
    w = c * NS + s
    pltpu.sync_copy(zeros_hbm, acc.at[pl.ds(s * RPT, RPT)])
    pltpu.sync_copy(dst_hbm.at[w], dst_v)
    pltpu.sync_copy(ones_hbm, ones_v)
    plsc.subcore_barrier()

    def body(b, carry):
        pltpu.sync_copy(ones_v, acc.at[dst_v.at[b]], add=True)
        return carry

    lax.fori_loop(0, NB, body, 0)
    plsc.subcore_barrier()
    pltpu.sync_copy(acc.at[pl.ds(s * RPT, RPT)],
                    out_hbm.at[c, pl.ds(s * RPT, RPT)])


@functools.partial(
    pl.kernel,
    out_type=jax.ShapeDtypeStruct((NC, NP, 128), jnp.float32),
    mesh=_MESH,
    name="sc_propagate",
    scratch_types=[
        pltpu.VMEM((NB, K), jnp.int32),
        pltpu.VMEM((NB, K), jnp.int32),
        pltpu.VMEM((2, K, 128), jnp.float32),
        pltpu.VMEM_SHARED((NP, 128), jnp.float32),
        pltpu.SemaphoreType.DMA,
        pltpu.SemaphoreType.DMA,
    ],
)
def _prop_kernel(g_hbm, src_hbm, dst_hbm, zeros_hbm, out_hbm,
                 src_v, dst_v, rows_v, acc, sem0, sem1):
    c = lax.axis_index("c")
    s = lax.axis_index("s")
    w = c * NS + s
    pltpu.sync_copy(zeros_hbm, acc.at[pl.ds(s * RPT, RPT)])
    pltpu.sync_copy(src_hbm.at[w], src_v)
    pltpu.sync_copy(dst_hbm.at[w], dst_v)
    plsc.subcore_barrier()

    sems = (sem0, sem1)
    pltpu.async_copy(g_hbm.at[src_v.at[0]], rows_v.at[0], sems[0])
    pltpu.async_copy(g_hbm.at[src_v.at[1]], rows_v.at[1], sems[1])

    def body(i, carry):
        for buf in range(2):
            b = i * 2 + buf
            pltpu.make_async_copy(
                g_hbm.at[src_v.at[b]], rows_v.at[buf], sems[buf]).wait()
            pltpu.sync_copy(rows_v.at[buf], acc.at[dst_v.at[b]], add=True)

            @pl.when(b + 2 < NB)
            def _():
                pltpu.async_copy(
                    g_hbm.at[src_v.at[b + 2]], rows_v.at[buf], sems[buf])
        return carry

    lax.fori_loop(0, NB // 2, body, 0)
    plsc.subcore_barrier()
    pltpu.sync_copy(acc.at[pl.ds(s * RPT, RPT)],
                    out_hbm.at[c, pl.ds(s * RPT, RPT)])


def _row_spec(shape):
    if len(shape) == 2:
        return pl.BlockSpec((BN, shape[1]), lambda i: (i, 0))
    return pl.BlockSpec((shape[0], BN, shape[2]), lambda i: (0, i, 0))


def _const_spec(shape):
    return pl.BlockSpec(shape, lambda i: (0,) * len(shape))


def _t1_body(cnt_ref, x_ref, dinv_ref, g0a_ref, g0b_ref):
    cnt = cnt_ref[...]
    deg = 1.0 + cnt[0, :, 0] + cnt[1, :, 0]
    dinv = lax.rsqrt(deg)[:, None]
    dinv_ref[...] = dinv
    g = dinv * x_ref[...]
    g0a_ref[...] = g[:, :128]
    g0b_ref[...] = g[:, 128:]


def _t2_body(p0_ref, p1_ref, g0a_ref, g0b_ref, dinv_ref, w1_ref, b1_ref,
             o0, o1, o2, o3):
    dinv = dinv_ref[...]
    qa = dinv * (p0_ref[0] + p0_ref[1] + g0a_ref[...])
    qb = dinv * (p1_ref[0] + p1_ref[1] + g0b_ref[...])
    q = jnp.concatenate([qa, qb], axis=1)
    h = jnp.dot(q, w1_ref[...], preferred_element_type=jnp.float32)
    h = jnp.maximum(h + b1_ref[...], 0.0)
    g = dinv * h
    o0[...] = g[:, 0:128]
    o1[...] = g[:, 128:256]
    o2[...] = g[:, 256:384]
    o3[...] = g[:, 384:512]


def _t3_body(p0_ref, p1_ref, p2_ref, p3_ref, g0_ref, g1_ref, g2_ref, g3_ref,
             dinv_ref, w2_ref, b2_ref, w3_ref, o0, o1):
    dinv = dinv_ref[...]
    cols = []
    for p_ref, g_ref in ((p0_ref, g0_ref), (p1_ref, g1_ref),
                         (p2_ref, g2_ref), (p3_ref, g3_ref)):
        cols.append(dinv * (p_ref[0] + p_ref[1] + g_ref[...]))
    q = jnp.concatenate(cols, axis=1)
    h = jnp.dot(q, w2_ref[...], preferred_element_type=jnp.float32)
    h = jnp.maximum(h + b2_ref[...], 0.0)
    y = jnp.dot(h, w3_ref[...], preferred_element_type=jnp.float32)
    g = dinv * y
    o0[...] = g[:, :128]
    o1[...] = g[:, 128:]


def _t4_body(p0_ref, p1_ref, g0_ref, g1_ref, dinv_ref, b3_ref, out_ref):
    dinv = dinv_ref[...]
    qa = dinv * (p0_ref[0] + p0_ref[1] + g0_ref[...])
    qb = dinv * (p1_ref[0] + p1_ref[1] + g1_ref[...])
    out_ref[...] = jnp.concatenate([qa, qb], axis=1) + b3_ref[...]


def _tc_call(body, in_arrays, out_shapes):
    return pl.pallas_call(
        body,
        grid=(GRID,),
        in_specs=[_row_spec(a.shape) if a.shape[len(a.shape) - 2 if
                  len(a.shape) == 2 else 1] in (N, NP) else _const_spec(a.shape)
                  for a in in_arrays],
        out_specs=[_row_spec(s.shape) for s in out_shapes],
        out_shape=out_shapes,
    )(*in_arrays)


def kernel(x, edge_index, W1, b1, W2, b2, W3, b3):
    src = edge_index[0].astype(jnp.int32)
    dst = edge_index[1].astype(jnp.int32)
    src3 = src.reshape(NW, NB, K)
    dst3 = dst.reshape(NW, NB, K)
    ones128 = jnp.ones((K, 128), jnp.float32)
    zeros128 = jnp.zeros((RPT, 128), jnp.float32)

    cnt = _deg_kernel(dst3, ones128, zeros128)

    f32 = jnp.float32
    dinv, g0a, g0b = _tc_call(
        _t1_body, [cnt, x],
        [jax.ShapeDtypeStruct((N, 1), f32),
         jax.ShapeDtypeStruct((N, 128), f32),
         jax.ShapeDtypeStruct((N, 128), f32)])

    p0 = _prop_kernel(g0a, src3, dst3, zeros128)
    p1 = _prop_kernel(g0b, src3, dst3, zeros128)

    g1 = _tc_call(
        _t2_body, [p0, p1, g0a, g0b, dinv, W1, b1.reshape(1, HID)],
        [jax.ShapeDtypeStruct((N, 128), f32)] * 4)

    q = [_prop_kernel(g, src3, dst3, zeros128) for g in g1]

    g2 = _tc_call(
        _t3_body, [*q, *g1, dinv, W2, b2.reshape(1, HID), W3],
        [jax.ShapeDtypeStruct((N, 128), f32)] * 2)

    r = [_prop_kernel(g, src3, dst3, zeros128) for g in g2]

    out, = _tc_call(
        _t4_body, [*r, *g2, dinv, b3.reshape(1, OUT_DIM)],
        [jax.ShapeDtypeStruct((N, OUT_DIM), f32)])
    return out


# speedup vs baseline: 16.4006x; 1.0742x over previous
import functools

import jax
import jax.numpy as jnp
from jax import lax
from jax.experimental import pallas as pl
from jax.experimental.pallas import tpu as pltpu
from jax.experimental.pallas import tpu_sc as plsc

N = 10000
E = 160000
IN_DIM = 256
HID = 512
OUT_DIM = 256

NC, NS = 2, 16
NW = NC * NS
EPW = E // NW
K = 100
NB = EPW // K
NP = 10240
RPT = NP // NS
ZR = 64
BN = 1000
GRID = N // BN

_MESH = plsc.VectorSubcoreMesh(
    core_axis_name="c", subcore_axis_name="s", num_cores=NC, num_subcores=NS)


def _make_prop(n_chunks, gather):
    def body(g_hbm, src_hbm, dst_hbm, zeros_hbm, out_hbm,
             src_v, dst_v, rows_v, zeros_v, acc, sem0, sem1):
        c = lax.axis_index("c")
        s = lax.axis_index("s")
        w = c * NS + s
        pltpu.sync_copy(zeros_hbm, zeros_v)
        if gather:
            pltpu.sync_copy(src_hbm.at[w], src_v)
        else:
            pltpu.sync_copy(g_hbm, rows_v.at[0])
        pltpu.sync_copy(dst_hbm.at[w], dst_v)
        sems = (sem0, sem1)

        for ch in range(n_chunks):
            for z in range(RPT // ZR):
                pltpu.sync_copy(zeros_v, acc.at[pl.ds(s * RPT + z * ZR, ZR)])
            plsc.subcore_barrier()

            if gather:
                gch = g_hbm.at[ch]
                pltpu.async_copy(gch.at[src_v.at[0]], rows_v.at[0], sems[0])
                pltpu.async_copy(gch.at[src_v.at[1]], rows_v.at[1], sems[1])

                def bb(i, carry):
                    for buf in range(2):
                        b = i * 2 + buf
                        pltpu.make_async_copy(
                            gch.at[src_v.at[b]], rows_v.at[buf],
                            sems[buf]).wait()
                        pltpu.sync_copy(rows_v.at[buf],
                                        acc.at[dst_v.at[b]], add=True)

                        @pl.when(b + 2 < NB)
                        def _():
                            pltpu.async_copy(gch.at[src_v.at[b + 2]],
                                             rows_v.at[buf], sems[buf])
                    return carry

                lax.fori_loop(0, NB // 2, bb, 0)
            else:
                def bb(i, carry):
                    pltpu.sync_copy(rows_v.at[0], acc.at[dst_v.at[i]],
                                    add=True)
                    return carry

                lax.fori_loop(0, NB, bb, 0)
            plsc.subcore_barrier()
            pltpu.sync_copy(acc.at[pl.ds(s * RPT, RPT)],
                            out_hbm.at[ch, c, pl.ds(s * RPT, RPT)])
            if ch + 1 < n_chunks:
                plsc.subcore_barrier()

    return pl.kernel(
        body,
        out_type=jax.ShapeDtypeStruct((n_chunks, NC, NP, 128), jnp.float32),
        mesh=_MESH,
        name="sc_propagate" if gather else "sc_degree",
        scratch_types=[
            pltpu.VMEM((NB, K), jnp.int32),
            pltpu.VMEM((NB, K), jnp.int32),
            pltpu.VMEM((2, K, 128), jnp.float32),
            pltpu.VMEM((ZR, 128), jnp.float32),
            pltpu.VMEM_SHARED((NP, 128), jnp.float32),
            pltpu.SemaphoreType.DMA,
            pltpu.SemaphoreType.DMA,
        ],
    )


_deg_kernel = _make_prop(1, gather=False)
_prop2 = _make_prop(2, gather=True)
_prop4 = _make_prop(4, gather=True)


def _t1_body(cnt_ref, x_ref, dinv_ref, g0_ref):
    cnt = cnt_ref[...]
    deg = 1.0 + cnt[0, 0, :, 0] + cnt[0, 1, :, 0]
    dinv = lax.rsqrt(deg)[:, None]
    dinv_ref[...] = dinv
    g = dinv * x_ref[...]
    g0_ref[0] = g[:, :128]
    g0_ref[1] = g[:, 128:]


def _t2_body(p_ref, g0_ref, dinv_ref, w1_ref, b1_ref, g1_ref):
    dinv = dinv_ref[...]
    qa = dinv * (p_ref[0, 0] + p_ref[0, 1] + g0_ref[0])
    qb = dinv * (p_ref[1, 0] + p_ref[1, 1] + g0_ref[1])
    q = jnp.concatenate([qa, qb], axis=1)
    h = jnp.dot(q, w1_ref[...], preferred_element_type=jnp.float32)
    h = jnp.maximum(h + b1_ref[...], 0.0)
    g = dinv * h
    for c in range(4):
        g1_ref[c] = g[:, 128 * c:128 * (c + 1)]


def _t3_body(p_ref, g1_ref, dinv_ref, w2_ref, b2_ref, w3_ref, g2_ref):
    dinv = dinv_ref[...]
    cols = [dinv * (p_ref[c, 0] + p_ref[c, 1] + g1_ref[c]) for c in range(4)]
    q = jnp.concatenate(cols, axis=1)
    h = jnp.dot(q, w2_ref[...], preferred_element_type=jnp.float32)
    h = jnp.maximum(h + b2_ref[...], 0.0)
    y = jnp.dot(h, w3_ref[...], preferred_element_type=jnp.float32)
    g = dinv * y
    g2_ref[0] = g[:, :128]
    g2_ref[1] = g[:, 128:]


def _t4_body(p_ref, g2_ref, dinv_ref, b3_ref, out_ref):
    dinv = dinv_ref[...]
    qa = dinv * (p_ref[0, 0] + p_ref[0, 1] + g2_ref[0])
    qb = dinv * (p_ref[1, 0] + p_ref[1, 1] + g2_ref[1])
    out_ref[...] = jnp.concatenate([qa, qb], axis=1) + b3_ref[...]


def _spec(shape):
    nd = len(shape)
    if shape[nd - 2] in (N, NP):
        blk = shape[:nd - 2] + (BN, shape[nd - 1])
        idx = (lambda i: (0,) * (nd - 2) + (i, 0))
        return pl.BlockSpec(blk, idx)
    return pl.BlockSpec(shape, lambda i: (0,) * nd)


def _tc_call(body, in_arrays, out_shapes):
    outs = pl.pallas_call(
        body,
        grid=(GRID,),
        in_specs=[_spec(a.shape) for a in in_arrays],
        out_specs=[_spec(s.shape) for s in out_shapes],
        out_shape=out_shapes,
    )(*in_arrays)
    return outs


def kernel(x, edge_index, W1, b1, W2, b2, W3, b3):
    src = edge_index[0].astype(jnp.int32)
    dst = edge_index[1].astype(jnp.int32)
    src3 = src.reshape(NW, NB, K)
    dst3 = dst.reshape(NW, NB, K)
    ones = jnp.ones((K, 128), jnp.float32)
    zeros = jnp.zeros((ZR, 128), jnp.float32)
    f32 = jnp.float32

    cnt = _deg_kernel(ones, src3, dst3, zeros)

    dinv, g0 = _tc_call(
        _t1_body, [cnt, x],
        [jax.ShapeDtypeStruct((N, 1), f32),
         jax.ShapeDtypeStruct((2, N, 128), f32)])

    p1 = _prop2(g0, src3, dst3, zeros)
    g1, = _tc_call(
        _t2_body, [p1, g0, dinv, W1, b1.reshape(1, HID)],
        [jax.ShapeDtypeStruct((4, N, 128), f32)])

    p2 = _prop4(g1, src3, dst3, zeros)
    g2, = _tc_call(
        _t3_body, [p2, g1, dinv, W2, b2.reshape(1, HID), W3],
        [jax.ShapeDtypeStruct((2, N, 128), f32)])

    p3 = _prop2(g2, src3, dst3, zeros)
    out, = _tc_call(
        _t4_body, [p3, g2, dinv, b3.reshape(1, OUT_DIM)],
        [jax.ShapeDtypeStruct((N, OUT_DIM), f32)])
    return out
